# Initial kernel scaffold; baseline (speedup 1.0000x reference)
import jax
import jax.numpy as jnp
from jax.experimental import pallas as pl


def kernel(z, A, B_mat, Z_share, Z_auth):
    raise NotImplementedError("write your pallas kernel here")


# speedup vs baseline: 3.4498x; 3.4498x over previous
import jax
import jax.numpy as jnp
from jax.experimental import pallas as pl
from jax.experimental.pallas import tpu as pltpu

DIM = 1024
HEADS = 16
EXPERTS = 8
BETA = 0.5
TOPK = 3
D_H = DIM // HEADS
R_H = 8
CAT = EXPERTS * R_H

TB = 512


def _gate(zr):
    m = jnp.max(zr, axis=-1, keepdims=True)
    e = jnp.exp(zr - m)
    p = e / jnp.sum(e, axis=-1, keepdims=True)
    pa = p[:, :, None]
    pb = p[:, None, :]
    ia = jax.lax.broadcasted_iota(jnp.int32, (HEADS, EXPERTS, EXPERTS), 1)
    ib = jax.lax.broadcasted_iota(jnp.int32, (HEADS, EXPERTS, EXPERTS), 2)
    beats = (pb > pa) | ((pb == pa) & (ib < ia))
    rank = jnp.sum(beats.astype(jnp.float32), axis=2)
    sel = jnp.where(rank < float(TOPK), p, 0.0)
    return sel / (jnp.sum(sel, axis=-1, keepdims=True) + 1e-8)


def _weights_kernel(zs_ref, za_ref, acat_ref, bcat_ref, w_ref):
    wtot = _gate(zs_ref[...]) + _gate(za_ref[...])
    re_ = jax.lax.broadcasted_iota(jnp.int32, (EXPERTS, CAT), 0)
    rc = jax.lax.broadcasted_iota(jnp.int32, (EXPERTS, CAT), 1)
    sel_mat = (rc // R_H == re_).astype(jnp.float32)
    wrep = jnp.dot(wtot, sel_mat, preferred_element_type=jnp.float32)
    ii = jax.lax.broadcasted_iota(jnp.int32, (D_H, D_H), 0)
    jj = jax.lax.broadcasted_iota(jnp.int32, (D_H, D_H), 1)
    eye = (ii == jj).astype(jnp.float32)
    acat = acat_ref[...]
    bcat = bcat_ref[...]
    for h in range(HEADS):
        aw = acat * wrep[h:h + 1, :]
        w_ref[h] = eye + BETA * jnp.dot(aw, bcat,
                                        preferred_element_type=jnp.float32)


def _apply_kernel(z_ref, w_ref, out_ref):
    z = z_ref[...]
    for h in range(HEADS):
        sl = slice(h * D_H, (h + 1) * D_H)
        out_ref[:, sl] = jnp.dot(z[:, sl], w_ref[h],
                                 preferred_element_type=jnp.float32)


def kernel(z, A, B_mat, Z_share, Z_auth):
    batch = z.shape[0]
    acat = jnp.transpose(A, (1, 0, 2)).reshape(D_H, CAT)
    bcat = B_mat.reshape(CAT, D_H)

    w = pl.pallas_call(
        _weights_kernel,
        out_shape=jax.ShapeDtypeStruct((HEADS, D_H, D_H), jnp.float32),
    )(Z_share, Z_auth, acat, bcat)

    out = pl.pallas_call(
        _apply_kernel,
        grid=(batch // TB,),
        in_specs=[
            pl.BlockSpec((TB, DIM), lambda i: (i, 0)),
            pl.BlockSpec((HEADS, D_H, D_H), lambda i: (0, 0, 0)),
        ],
        out_specs=pl.BlockSpec((TB, DIM), lambda i: (i, 0)),
        out_shape=jax.ShapeDtypeStruct((batch, DIM), jnp.float32),
        compiler_params=pltpu.CompilerParams(
            dimension_semantics=("parallel",),
        ),
    )(z, w)
    return out


# speedup vs baseline: 4.5520x; 1.3195x over previous
import jax
import jax.numpy as jnp
from jax.experimental import pallas as pl
from jax.experimental.pallas import tpu as pltpu

DIM = 1024
HEADS = 16
EXPERTS = 8
BETA = 0.5
TOPK = 3
D_H = DIM // HEADS
R_H = 8
CAT = EXPERTS * R_H

TB = 512


def _gate(zr):
    m = jnp.max(zr, axis=-1, keepdims=True)
    e = jnp.exp(zr - m)
    p = e / jnp.sum(e, axis=-1, keepdims=True)
    pa = p[:, :, None]
    pb = p[:, None, :]
    ia = jax.lax.broadcasted_iota(jnp.int32, (HEADS, EXPERTS, EXPERTS), 1)
    ib = jax.lax.broadcasted_iota(jnp.int32, (HEADS, EXPERTS, EXPERTS), 2)
    beats = (pb > pa) | ((pb == pa) & (ib < ia))
    rank = jnp.sum(beats.astype(jnp.float32), axis=2)
    sel = jnp.where(rank < float(TOPK), p, 0.0)
    return sel / (jnp.sum(sel, axis=-1, keepdims=True) + 1e-8)


def _weights_kernel(zs_ref, za_ref, acat_ref, bcat_ref, w_ref):
    wtot = _gate(zs_ref[...]) + _gate(za_ref[...])
    re_ = jax.lax.broadcasted_iota(jnp.int32, (EXPERTS, CAT), 0)
    rc = jax.lax.broadcasted_iota(jnp.int32, (EXPERTS, CAT), 1)
    sel_mat = (rc // R_H == re_).astype(jnp.float32)
    wrep = jnp.dot(wtot, sel_mat, preferred_element_type=jnp.float32)
    acat = acat_ref[...]
    bcat = bcat_ref[...]
    zero = jnp.zeros((D_H, D_H), dtype=jnp.bfloat16)
    for p in range(HEADS // 2):
        ma = (BETA * jnp.dot(acat * wrep[2 * p:2 * p + 1, :], bcat,
                             preferred_element_type=jnp.float32)
              ).astype(jnp.bfloat16)
        mb = (BETA * jnp.dot(acat * wrep[2 * p + 1:2 * p + 2, :], bcat,
                             preferred_element_type=jnp.float32)
              ).astype(jnp.bfloat16)
        w_ref[p, 0:D_H, 0:D_H] = ma
        w_ref[p, 0:D_H, D_H:2 * D_H] = zero
        w_ref[p, D_H:2 * D_H, 0:D_H] = zero
        w_ref[p, D_H:2 * D_H, D_H:2 * D_H] = mb


def _apply_kernel(z_ref, w_ref, out_ref):
    z = z_ref[...]
    zb = z.astype(jnp.bfloat16)
    for p in range(HEADS // 2):
        sl = slice(p * 2 * D_H, (p + 1) * 2 * D_H)
        out_ref[:, sl] = z[:, sl] + jnp.dot(zb[:, sl], w_ref[p],
                                            preferred_element_type=jnp.float32)


def kernel(z, A, B_mat, Z_share, Z_auth):
    batch = z.shape[0]
    acat = jnp.transpose(A, (1, 0, 2)).reshape(D_H, CAT)
    bcat = B_mat.reshape(CAT, D_H)

    w = pl.pallas_call(
        _weights_kernel,
        out_shape=jax.ShapeDtypeStruct((HEADS // 2, 2 * D_H, 2 * D_H),
                                       jnp.bfloat16),
    )(Z_share, Z_auth, acat, bcat)

    out = pl.pallas_call(
        _apply_kernel,
        grid=(batch // TB,),
        in_specs=[
            pl.BlockSpec((TB, DIM), lambda i: (i, 0)),
            pl.BlockSpec((HEADS // 2, 2 * D_H, 2 * D_H), lambda i: (0, 0, 0)),
        ],
        out_specs=pl.BlockSpec((TB, DIM), lambda i: (i, 0)),
        out_shape=jax.ShapeDtypeStruct((batch, DIM), jnp.float32),
        compiler_params=pltpu.CompilerParams(
            dimension_semantics=("parallel",),
        ),
    )(z, w)
    return out


# speedup vs baseline: 5.0055x; 1.0996x over previous
import jax
import jax.numpy as jnp
from jax.experimental import pallas as pl
from jax.experimental.pallas import tpu as pltpu

DIM = 1024
HEADS = 16
EXPERTS = 8
BETA = 0.5
TOPK = 3
D_H = DIM // HEADS
R_H = 8
CAT = EXPERTS * R_H

TB = 1024


def _gate(zr):
    m = jnp.max(zr, axis=-1, keepdims=True)
    e = jnp.exp(zr - m)
    p = e / jnp.sum(e, axis=-1, keepdims=True)
    pa = p[:, :, None]
    pb = p[:, None, :]
    ia = jax.lax.broadcasted_iota(jnp.int32, (HEADS, EXPERTS, EXPERTS), 1)
    ib = jax.lax.broadcasted_iota(jnp.int32, (HEADS, EXPERTS, EXPERTS), 2)
    beats = (pb > pa) | ((pb == pa) & (ib < ia))
    rank = jnp.sum(beats.astype(jnp.float32), axis=2)
    sel = jnp.where(rank < float(TOPK), p, 0.0)
    return sel / (jnp.sum(sel, axis=-1, keepdims=True) + 1e-8)


def _weights_kernel(zs_ref, za_ref, acat_ref, bcat_ref, w_ref):
    wtot = _gate(zs_ref[...]) + _gate(za_ref[...])
    re_ = jax.lax.broadcasted_iota(jnp.int32, (EXPERTS, CAT), 0)
    rc = jax.lax.broadcasted_iota(jnp.int32, (EXPERTS, CAT), 1)
    sel_mat = (rc // R_H == re_).astype(jnp.float32)
    wrep = jnp.dot(wtot, sel_mat, preferred_element_type=jnp.float32)
    acat = acat_ref[...]
    bcat = bcat_ref[...]
    zero = jnp.zeros((D_H, D_H), dtype=jnp.bfloat16)
    for p in range(HEADS // 2):
        ma = (BETA * jnp.dot(acat * wrep[2 * p:2 * p + 1, :], bcat,
                             preferred_element_type=jnp.float32)
              ).astype(jnp.bfloat16)
        mb = (BETA * jnp.dot(acat * wrep[2 * p + 1:2 * p + 2, :], bcat,
                             preferred_element_type=jnp.float32)
              ).astype(jnp.bfloat16)
        w_ref[p, 0:D_H, 0:D_H] = ma
        w_ref[p, 0:D_H, D_H:2 * D_H] = zero
        w_ref[p, D_H:2 * D_H, 0:D_H] = zero
        w_ref[p, D_H:2 * D_H, D_H:2 * D_H] = mb


def _apply_kernel(z_ref, w_ref, out_ref):
    z = z_ref[...]
    zb = z.astype(jnp.bfloat16)
    for p in range(HEADS // 2):
        sl = slice(p * 2 * D_H, (p + 1) * 2 * D_H)
        out_ref[:, sl] = z[:, sl] + jnp.dot(zb[:, sl], w_ref[p],
                                            preferred_element_type=jnp.float32)


def kernel(z, A, B_mat, Z_share, Z_auth):
    batch = z.shape[0]
    acat = jnp.transpose(A, (1, 0, 2)).reshape(D_H, CAT)
    bcat = B_mat.reshape(CAT, D_H)

    w = pl.pallas_call(
        _weights_kernel,
        out_shape=jax.ShapeDtypeStruct((HEADS // 2, 2 * D_H, 2 * D_H),
                                       jnp.bfloat16),
    )(Z_share, Z_auth, acat, bcat)

    out = pl.pallas_call(
        _apply_kernel,
        grid=(batch // TB,),
        in_specs=[
            pl.BlockSpec((TB, DIM), lambda i: (i, 0)),
            pl.BlockSpec((HEADS // 2, 2 * D_H, 2 * D_H), lambda i: (0, 0, 0)),
        ],
        out_specs=pl.BlockSpec((TB, DIM), lambda i: (i, 0)),
        out_shape=jax.ShapeDtypeStruct((batch, DIM), jnp.float32),
        compiler_params=pltpu.CompilerParams(
            dimension_semantics=("parallel",),
        ),
    )(z, w)
    return out


# speedup vs baseline: 5.1452x; 1.0279x over previous
import jax
import jax.numpy as jnp
from jax.experimental import pallas as pl
from jax.experimental.pallas import tpu as pltpu

DIM = 1024
HEADS = 16
EXPERTS = 8
BETA = 0.5
TOPK = 3
D_H = DIM // HEADS
R_H = 8
CAT = EXPERTS * R_H

TB = 2048


def _gate(zr):
    m = jnp.max(zr, axis=-1, keepdims=True)
    e = jnp.exp(zr - m)
    p = e / jnp.sum(e, axis=-1, keepdims=True)
    pa = p[:, :, None]
    pb = p[:, None, :]
    ia = jax.lax.broadcasted_iota(jnp.int32, (HEADS, EXPERTS, EXPERTS), 1)
    ib = jax.lax.broadcasted_iota(jnp.int32, (HEADS, EXPERTS, EXPERTS), 2)
    beats = (pb > pa) | ((pb == pa) & (ib < ia))
    rank = jnp.sum(beats.astype(jnp.float32), axis=2)
    sel = jnp.where(rank < float(TOPK), p, 0.0)
    return sel / (jnp.sum(sel, axis=-1, keepdims=True) + 1e-8)


def _weights_kernel(zs_ref, za_ref, acat_ref, bcat_ref, w_ref):
    wtot = _gate(zs_ref[...]) + _gate(za_ref[...])
    re_ = jax.lax.broadcasted_iota(jnp.int32, (EXPERTS, CAT), 0)
    rc = jax.lax.broadcasted_iota(jnp.int32, (EXPERTS, CAT), 1)
    sel_mat = (rc // R_H == re_).astype(jnp.float32)
    wrep = jnp.dot(wtot, sel_mat, preferred_element_type=jnp.float32)
    acat = acat_ref[...]
    bcat = bcat_ref[...]
    zero = jnp.zeros((D_H, D_H), dtype=jnp.bfloat16)
    for p in range(HEADS // 2):
        ma = (BETA * jnp.dot(acat * wrep[2 * p:2 * p + 1, :], bcat,
                             preferred_element_type=jnp.float32)
              ).astype(jnp.bfloat16)
        mb = (BETA * jnp.dot(acat * wrep[2 * p + 1:2 * p + 2, :], bcat,
                             preferred_element_type=jnp.float32)
              ).astype(jnp.bfloat16)
        w_ref[p, 0:D_H, 0:D_H] = ma
        w_ref[p, 0:D_H, D_H:2 * D_H] = zero
        w_ref[p, D_H:2 * D_H, 0:D_H] = zero
        w_ref[p, D_H:2 * D_H, D_H:2 * D_H] = mb


def _apply_kernel(z_ref, w_ref, out_ref):
    z = z_ref[...]
    zb = z.astype(jnp.bfloat16)
    for p in range(HEADS // 2):
        sl = slice(p * 2 * D_H, (p + 1) * 2 * D_H)
        out_ref[:, sl] = z[:, sl] + jnp.dot(zb[:, sl], w_ref[p],
                                            preferred_element_type=jnp.float32)


def kernel(z, A, B_mat, Z_share, Z_auth):
    batch = z.shape[0]
    acat = jnp.transpose(A, (1, 0, 2)).reshape(D_H, CAT)
    bcat = B_mat.reshape(CAT, D_H)

    w = pl.pallas_call(
        _weights_kernel,
        out_shape=jax.ShapeDtypeStruct((HEADS // 2, 2 * D_H, 2 * D_H),
                                       jnp.bfloat16),
    )(Z_share, Z_auth, acat, bcat)

    out = pl.pallas_call(
        _apply_kernel,
        grid=(batch // TB,),
        in_specs=[
            pl.BlockSpec((TB, DIM), lambda i: (i, 0)),
            pl.BlockSpec((HEADS // 2, 2 * D_H, 2 * D_H), lambda i: (0, 0, 0)),
        ],
        out_specs=pl.BlockSpec((TB, DIM), lambda i: (i, 0)),
        out_shape=jax.ShapeDtypeStruct((batch, DIM), jnp.float32),
        compiler_params=pltpu.CompilerParams(
            dimension_semantics=("parallel",),
        ),
    )(z, w)
    return out
